# baseline (speedup 1.0000x reference)
import functools

import numpy as np
import jax
import jax.numpy as jnp
from jax import lax
from jax.experimental import pallas as pl
from jax.experimental.pallas import tpu as pltpu
from jax.experimental.pallas import tpu_sc as plsc

_CONTEXT_WINDOW = 200
_EMBEDDING_DIM = 64
_LANES = 16


def _pe_np(context_window, embedding_dim):
    pos = np.arange(context_window, dtype=np.float32)[:, None]
    i = np.arange(embedding_dim, dtype=np.float32)[None, :]
    angle = pos / np.power(10000.0, i / embedding_dim)
    pe = np.where((np.arange(embedding_dim)[None, :] % 2) == 0,
                  np.sin(angle), np.cos(angle))
    return pe.astype(np.float32)


@functools.lru_cache(maxsize=None)
def _build_sc_call(B, D, b_per_w):
    mesh = plsc.VectorSubcoreMesh(core_axis_name="c", subcore_axis_name="s")
    info = plsc.get_sparse_core_info()
    nc = info.num_cores

    @functools.partial(
        pl.kernel,
        mesh=mesh,
        out_type=jax.ShapeDtypeStruct((B, D), jnp.float32),
        scratch_types=[
            pltpu.VMEM((b_per_w,), jnp.int32),
            pltpu.VMEM((b_per_w, D), jnp.float32),
            pltpu.VMEM((b_per_w, D), jnp.float32),
            pltpu.SemaphoreType.DMA,
        ],
        compiler_params=pltpu.CompilerParams(use_tc_tiling_on_sc=False),
    )
    def sc_embed(x_hbm, table_hbm, pe_hbm, out_hbm, idx_v, rows_v, pe_v, sem):
        wid = lax.axis_index("s") * nc + lax.axis_index("c")
        base = wid * b_per_w
        pltpu.sync_copy(x_hbm.at[pl.ds(base, b_per_w)], idx_v)
        gather = pltpu.async_copy(table_hbm.at[idx_v], rows_v, sem)
        pltpu.sync_copy(pe_hbm.at[pl.ds(base, b_per_w)], pe_v)
        gather.wait()
        for i in range(b_per_w):
            for j in range(D // _LANES):
                s = pl.ds(j * _LANES, _LANES)
                rows_v[i, s] = rows_v[i, s] + pe_v[i, s]
        pltpu.sync_copy(rows_v, out_hbm.at[pl.ds(base, b_per_w)])

    return sc_embed


def kernel(x, table):
    B = 256
    D = _EMBEDDING_DIM
    pe = np.zeros((B, D), dtype=np.float32)
    pe[:_CONTEXT_WINDOW] = _pe_np(_CONTEXT_WINDOW, D)
    x_pad = jnp.pad(x.astype(jnp.int32), (0, B - _CONTEXT_WINDOW))
    out = _build_sc_call(B, D, B // 32)(x_pad, table, jnp.asarray(pe))
    return out[:_CONTEXT_WINDOW]


# speedup vs baseline: 1.0030x; 1.0030x over previous
import functools

import numpy as np
import jax
import jax.numpy as jnp
from jax import lax
from jax.experimental import pallas as pl
from jax.experimental.pallas import tpu as pltpu
from jax.experimental.pallas import tpu_sc as plsc

_CONTEXT_WINDOW = 200
_EMBEDDING_DIM = 64
_LANES = 16


def _pe_np(context_window, embedding_dim):
    pos = np.arange(context_window, dtype=np.float32)[:, None]
    i = np.arange(embedding_dim, dtype=np.float32)[None, :]
    angle = pos / np.power(10000.0, i / embedding_dim)
    pe = np.where((np.arange(embedding_dim)[None, :] % 2) == 0,
                  np.sin(angle), np.cos(angle))
    return pe.astype(np.float32)


@functools.lru_cache(maxsize=None)
def _build_sc_call(B, D, b_per_w):
    mesh = plsc.VectorSubcoreMesh(core_axis_name="c", subcore_axis_name="s")
    info = plsc.get_sparse_core_info()
    nc = info.num_cores
    n_active = B // b_per_w

    @functools.partial(
        pl.kernel,
        mesh=mesh,
        out_type=jax.ShapeDtypeStruct((B, D), jnp.float32),
        scratch_types=[
            pltpu.VMEM((b_per_w,), jnp.int32),
            pltpu.VMEM((b_per_w, D), jnp.float32),
            pltpu.VMEM((b_per_w, D), jnp.float32),
            pltpu.SemaphoreType.DMA,
        ],
        compiler_params=pltpu.CompilerParams(use_tc_tiling_on_sc=False),
    )
    def sc_embed(x_hbm, table_hbm, pe_hbm, out_hbm, idx_v, rows_v, pe_v, sem):
        wid = lax.axis_index("s") * nc + lax.axis_index("c")

        @pl.when(wid < n_active)
        def _():
            base = wid * b_per_w
            pltpu.sync_copy(x_hbm.at[pl.ds(base, b_per_w)], idx_v)
            gather = pltpu.async_copy(table_hbm.at[idx_v], rows_v, sem)
            pltpu.sync_copy(pe_hbm.at[pl.ds(base, b_per_w)], pe_v)
            gather.wait()
            for i in range(b_per_w):
                for j in range(D // _LANES):
                    s = pl.ds(j * _LANES, _LANES)
                    rows_v[i, s] = rows_v[i, s] + pe_v[i, s]
            pltpu.sync_copy(rows_v, out_hbm.at[pl.ds(base, b_per_w)])

    return sc_embed


def kernel(x, table):
    pe = _pe_np(_CONTEXT_WINDOW, _EMBEDDING_DIM)
    return _build_sc_call(_CONTEXT_WINDOW, _EMBEDDING_DIM, 8)(
        x, table, jnp.asarray(pe))


# speedup vs baseline: 1.7367x; 1.7315x over previous
import functools

import numpy as np
import jax
import jax.numpy as jnp
from jax import lax
from jax.experimental import pallas as pl
from jax.experimental.pallas import tpu as pltpu
from jax.experimental.pallas import tpu_sc as plsc

_CONTEXT_WINDOW = 200
_EMBEDDING_DIM = 64
_LANES = 16


def _pe_np(context_window, embedding_dim):
    pos = np.arange(context_window, dtype=np.float32)[:, None]
    i = np.arange(embedding_dim, dtype=np.float32)[None, :]
    angle = pos / np.power(10000.0, i / embedding_dim)
    pe = np.where((np.arange(embedding_dim)[None, :] % 2) == 0,
                  np.sin(angle), np.cos(angle))
    return pe.astype(np.float32)


@functools.lru_cache(maxsize=None)
def _build_sc_call(B, D, b_per_w):
    mesh = plsc.VectorSubcoreMesh(core_axis_name="c", subcore_axis_name="s")
    info = plsc.get_sparse_core_info()
    nc = info.num_cores
    n_active = B // b_per_w

    @functools.partial(
        pl.kernel,
        mesh=mesh,
        out_type=jax.ShapeDtypeStruct((B, D), jnp.float32),
        scratch_types=[
            pltpu.VMEM((_LANES,), jnp.int32),
            pltpu.VMEM((b_per_w, D), jnp.float32),
            pltpu.VMEM((b_per_w, D), jnp.float32),
            pltpu.SemaphoreType.DMA,
        ],
        compiler_params=pltpu.CompilerParams(needs_layout_passes=False),
    )
    def sc_embed(x_hbm, table_hbm, pe_hbm, out_hbm, idx_v, rows_v, pe_v, sem):
        wid = lax.axis_index("s") * nc + lax.axis_index("c")

        @pl.when(wid < n_active)
        def _():
            base = wid * b_per_w
            pltpu.sync_copy(x_hbm.at[pl.ds(base, b_per_w)],
                            idx_v.at[pl.ds(0, b_per_w)])
            idx_vec = idx_v[...]
            lane = lax.iota(jnp.int32, _LANES)
            copies = []
            for i in range(b_per_w):
                row = jnp.max(jnp.where(lane == i, idx_vec, 0))
                copies.append(pltpu.async_copy(
                    table_hbm.at[pl.ds(row, 1)], rows_v.at[pl.ds(i, 1)], sem))
            pltpu.sync_copy(pe_hbm.at[pl.ds(base, b_per_w)], pe_v)
            for c in copies:
                c.wait()
            for i in range(b_per_w):
                for j in range(D // _LANES):
                    s = pl.ds(j * _LANES, _LANES)
                    rows_v[i, s] = rows_v[i, s] + pe_v[i, s]
            pltpu.sync_copy(rows_v, out_hbm.at[pl.ds(base, b_per_w)])

    return sc_embed


def kernel(x, table):
    pe = _pe_np(_CONTEXT_WINDOW, _EMBEDDING_DIM)
    return _build_sc_call(_CONTEXT_WINDOW, _EMBEDDING_DIM, 8)(
        x, table, jnp.asarray(pe))


# speedup vs baseline: 24.4045x; 14.0521x over previous
import functools

import numpy as np
import jax
import jax.numpy as jnp
from jax import lax
from jax.experimental import pallas as pl
from jax.experimental.pallas import tpu as pltpu
from jax.experimental.pallas import tpu_sc as plsc

_CONTEXT_WINDOW = 200
_EMBEDDING_DIM = 64
_LANES = 16
_BLK = 128


def _pe_np(context_window, embedding_dim):
    pos = np.arange(context_window, dtype=np.float32)[:, None]
    i = np.arange(embedding_dim, dtype=np.float32)[None, :]
    angle = pos / np.power(10000.0, i / embedding_dim)
    pe = np.where((np.arange(embedding_dim)[None, :] % 2) == 0,
                  np.sin(angle), np.cos(angle))
    return pe.astype(np.float32)


@functools.lru_cache(maxsize=None)
def _build_sc_call(B, D, b_per_w):
    mesh = plsc.VectorSubcoreMesh(core_axis_name="c", subcore_axis_name="s")
    info = plsc.get_sparse_core_info()
    nc = info.num_cores
    n_active = B // b_per_w

    @functools.partial(
        pl.kernel,
        mesh=mesh,
        out_type=jax.ShapeDtypeStruct((B, D), jnp.float32),
        scratch_types=[
            pltpu.VMEM((_LANES,), jnp.int32),
            pltpu.VMEM((b_per_w, D, _BLK), jnp.float32),
            pltpu.VMEM((b_per_w, D), jnp.float32),
            pltpu.VMEM((b_per_w, D), jnp.float32),
            pltpu.SemaphoreType.DMA,
        ],
        compiler_params=pltpu.CompilerParams(needs_layout_passes=False),
    )
    def sc_embed(x_hbm, tab_t_hbm, pe_hbm, out_hbm, idx_v, blocks_v, rows_v,
                 pe_v, sem):
        wid = lax.axis_index("s") * nc + lax.axis_index("c")

        @pl.when(wid < n_active)
        def _():
            base = wid * b_per_w
            pltpu.sync_copy(x_hbm.at[pl.ds(base, b_per_w)],
                            idx_v.at[pl.ds(0, b_per_w)])
            idx_vec = idx_v[...]
            copies = []
            cols = []
            for i in range(b_per_w):
                row = idx_vec[i]
                col = row & (_BLK - 1)
                blk = pl.multiple_of(row - col, _BLK)
                cols.append(col)
                copies.append(pltpu.async_copy(
                    tab_t_hbm.at[:, pl.ds(blk, _BLK)], blocks_v.at[i], sem))
            pltpu.sync_copy(pe_hbm.at[pl.ds(base, b_per_w)], pe_v)
            for c in copies:
                c.wait()
            lane = lax.iota(jnp.int32, _LANES)
            for i in range(b_per_w):
                sel_i = jnp.full((_LANES,), i, jnp.int32)
                col_b = jnp.full((_LANES,), cols[i], jnp.int32)
                for j in range(D // _LANES):
                    s = pl.ds(j * _LANES, _LANES)
                    val = plsc.load_gather(
                        blocks_v, [sel_i, j * _LANES + lane, col_b])
                    rows_v[i, s] = val + pe_v[i, s]
            pltpu.sync_copy(rows_v, out_hbm.at[pl.ds(base, b_per_w)])

    return sc_embed


def kernel(x, table):
    pe = _pe_np(_CONTEXT_WINDOW, _EMBEDDING_DIM)
    return _build_sc_call(_CONTEXT_WINDOW, _EMBEDDING_DIM, 8)(
        x, table.T, jnp.asarray(pe))


# speedup vs baseline: 24.4126x; 1.0003x over previous
import functools

import numpy as np
import jax
import jax.numpy as jnp
from jax import lax
from jax.experimental import pallas as pl
from jax.experimental.pallas import tpu as pltpu
from jax.experimental.pallas import tpu_sc as plsc

_CONTEXT_WINDOW = 200
_EMBEDDING_DIM = 64
_LANES = 16
_BLK = 128


def _pe_np(context_window, embedding_dim):
    pos = np.arange(context_window, dtype=np.float32)[:, None]
    i = np.arange(embedding_dim, dtype=np.float32)[None, :]
    angle = pos / np.power(10000.0, i / embedding_dim)
    pe = np.where((np.arange(embedding_dim)[None, :] % 2) == 0,
                  np.sin(angle), np.cos(angle))
    return pe.astype(np.float32)


@functools.lru_cache(maxsize=None)
def _build_sc_call(B, D, b_per_w):
    mesh = plsc.VectorSubcoreMesh(core_axis_name="c", subcore_axis_name="s")
    info = plsc.get_sparse_core_info()
    nc = info.num_cores
    n_active = B // b_per_w

    @functools.partial(
        pl.kernel,
        mesh=mesh,
        out_type=jax.ShapeDtypeStruct((B, D), jnp.float32),
        scratch_types=[
            pltpu.VMEM((_LANES,), jnp.int32),
            pltpu.VMEM((b_per_w, D, _BLK), jnp.float32),
            pltpu.VMEM((b_per_w, D), jnp.float32),
            pltpu.VMEM((b_per_w * D,), jnp.float32),
            pltpu.SemaphoreType.DMA,
        ],
        compiler_params=pltpu.CompilerParams(needs_layout_passes=False),
    )
    def sc_embed(x_hbm, tab_t_hbm, pe_hbm, out_hbm, idx_v, blocks_v, rows_v,
                 pe_v, sem):
        wid = lax.axis_index("s") * nc + lax.axis_index("c")

        @pl.when(wid < n_active)
        def _():
            base = wid * b_per_w
            pltpu.sync_copy(x_hbm.at[pl.ds(base, b_per_w)],
                            idx_v.at[pl.ds(0, b_per_w)])
            idx_vec = idx_v[...]
            copies = []
            cols = []
            for i in range(b_per_w):
                row = idx_vec[i]
                col = row & (_BLK - 1)
                blk = pl.multiple_of(row - col, _BLK)
                cols.append(col)
                copies.append(pltpu.async_copy(
                    tab_t_hbm.at[:, pl.ds(blk, _BLK)], blocks_v.at[i], sem))
            pltpu.sync_copy(pe_hbm.at[pl.ds(base * D, b_per_w * D)], pe_v)
            for c in copies:
                c.wait()
            lane = lax.iota(jnp.int32, _LANES)
            for i in range(b_per_w):
                sel_i = jnp.full((_LANES,), i, jnp.int32)
                col_b = jnp.full((_LANES,), cols[i], jnp.int32)
                for j in range(D // _LANES):
                    s = pl.ds(j * _LANES, _LANES)
                    val = plsc.load_gather(
                        blocks_v, [sel_i, j * _LANES + lane, col_b])
                    rows_v[i, s] = val + pe_v[pl.ds(i * D + j * _LANES, _LANES)]
            pltpu.sync_copy(rows_v, out_hbm.at[pl.ds(base, b_per_w)])

    return sc_embed


def kernel(x, table):
    pe = _pe_np(_CONTEXT_WINDOW, _EMBEDDING_DIM).reshape(-1)
    return _build_sc_call(_CONTEXT_WINDOW, _EMBEDDING_DIM, 8)(
        x, table.T, jnp.asarray(pe))


# speedup vs baseline: 24.4437x; 1.0013x over previous
import functools

import numpy as np
import jax
import jax.numpy as jnp
from jax import lax
from jax.experimental import pallas as pl
from jax.experimental.pallas import tpu as pltpu
from jax.experimental.pallas import tpu_sc as plsc

_CONTEXT_WINDOW = 200
_EMBEDDING_DIM = 64
_LANES = 16
_BLK = 128


def _pe_np(context_window, embedding_dim):
    pos = np.arange(context_window, dtype=np.float32)[:, None]
    i = np.arange(embedding_dim, dtype=np.float32)[None, :]
    angle = pos / np.power(10000.0, i / embedding_dim)
    pe = np.where((np.arange(embedding_dim)[None, :] % 2) == 0,
                  np.sin(angle), np.cos(angle))
    return pe.astype(np.float32)


@functools.lru_cache(maxsize=None)
def _build_sc_call(B, D, b_per_w):
    mesh = plsc.VectorSubcoreMesh(core_axis_name="c", subcore_axis_name="s")
    info = plsc.get_sparse_core_info()
    nc = info.num_cores
    n_active = B // b_per_w

    @functools.partial(
        pl.kernel,
        mesh=mesh,
        out_type=jax.ShapeDtypeStruct((B * D,), jnp.float32),
        scratch_types=[
            pltpu.VMEM((_LANES,), jnp.int32),
            pltpu.VMEM((b_per_w, D, _BLK), jnp.float32),
            pltpu.VMEM((b_per_w * D,), jnp.float32),
            pltpu.VMEM((b_per_w * D,), jnp.float32),
            pltpu.SemaphoreType.DMA,
        ],
        compiler_params=pltpu.CompilerParams(needs_layout_passes=False),
    )
    def sc_embed(x_hbm, tab_t_hbm, pe_hbm, out_hbm, idx_v, blocks_v, rows_v,
                 pe_v, sem):
        wid = lax.axis_index("s") * nc + lax.axis_index("c")

        @pl.when(wid < n_active)
        def _():
            base = wid * b_per_w
            pltpu.sync_copy(x_hbm.at[pl.ds(base, b_per_w)],
                            idx_v.at[pl.ds(0, b_per_w)])
            idx_vec = idx_v[...]
            copies = []
            cols = []
            for i in range(b_per_w):
                row = idx_vec[i]
                col = row & (_BLK - 1)
                blk = pl.multiple_of(row - col, _BLK)
                cols.append(col)
                copies.append(pltpu.async_copy(
                    tab_t_hbm.at[:, pl.ds(blk, _BLK)], blocks_v.at[i], sem))
            pltpu.sync_copy(pe_hbm.at[pl.ds(base * D, b_per_w * D)], pe_v)
            for c in copies:
                c.wait()
            lane = lax.iota(jnp.int32, _LANES)
            for i in range(b_per_w):
                sel_i = jnp.full((_LANES,), i, jnp.int32)
                col_b = jnp.full((_LANES,), cols[i], jnp.int32)
                for j in range(D // _LANES):
                    s = pl.ds(i * D + j * _LANES, _LANES)
                    val = plsc.load_gather(
                        blocks_v, [sel_i, j * _LANES + lane, col_b])
                    rows_v[s] = val + pe_v[s]
            pltpu.sync_copy(rows_v, out_hbm.at[pl.ds(base * D, b_per_w * D)])

    return sc_embed


def kernel(x, table):
    pe = _pe_np(_CONTEXT_WINDOW, _EMBEDDING_DIM).reshape(-1)
    out = _build_sc_call(_CONTEXT_WINDOW, _EMBEDDING_DIM, 8)(
        x, table.T, jnp.asarray(pe))
    return out.reshape(_CONTEXT_WINDOW, _EMBEDDING_DIM)


# speedup vs baseline: 24.4514x; 1.0003x over previous
import functools

import numpy as np
import jax
import jax.numpy as jnp
from jax import lax
from jax.experimental import pallas as pl
from jax.experimental.pallas import tpu as pltpu
from jax.experimental.pallas import tpu_sc as plsc

_CONTEXT_WINDOW = 200
_EMBEDDING_DIM = 64
_LANES = 16
_BLK = 128


def _pe_np(context_window, embedding_dim):
    pos = np.arange(context_window, dtype=np.float32)[:, None]
    i = np.arange(embedding_dim, dtype=np.float32)[None, :]
    angle = pos / np.power(10000.0, i / embedding_dim)
    pe = np.where((np.arange(embedding_dim)[None, :] % 2) == 0,
                  np.sin(angle), np.cos(angle))
    return pe.astype(np.float32)


@functools.lru_cache(maxsize=None)
def _build_sc_call(B, D, b_per_w):
    mesh = plsc.VectorSubcoreMesh(core_axis_name="c", subcore_axis_name="s")
    info = plsc.get_sparse_core_info()
    nc = info.num_cores
    n_active = B // b_per_w

    @functools.partial(
        pl.kernel,
        mesh=mesh,
        out_type=jax.ShapeDtypeStruct((B * D,), jnp.float32),
        scratch_types=[
            pltpu.VMEM((2 * _LANES,), jnp.int32),
            pltpu.VMEM((b_per_w, D, _BLK), jnp.float32),
            pltpu.VMEM((b_per_w * D,), jnp.float32),
            pltpu.VMEM((b_per_w * D,), jnp.float32),
            pltpu.SemaphoreType.DMA,
        ],
        compiler_params=pltpu.CompilerParams(needs_layout_passes=False),
    )
    def sc_embed(x_hbm, tab_t_hbm, pe_hbm, out_hbm, idx_v, blocks_v, rows_v,
                 pe_v, sem):
        wid = lax.axis_index("s") * nc + lax.axis_index("c")

        @pl.when(wid < n_active)
        def _():
            base = wid * b_per_w
            pltpu.sync_copy(x_hbm.at[pl.ds(base, b_per_w)],
                            idx_v.at[pl.ds(0, b_per_w)])

            def _row_at(i):
                return idx_v[pl.ds(i, _LANES)][0]

            def issue(i, carry):
                row = _row_at(i)
                col = row & (_BLK - 1)
                blk = pl.multiple_of(row - col, _BLK)
                pltpu.async_copy(
                    tab_t_hbm.at[:, pl.ds(blk, _BLK)], blocks_v.at[i], sem)
                return carry

            lax.fori_loop(0, b_per_w, issue, 0)
            pltpu.sync_copy(pe_hbm.at[pl.ds(base * D, b_per_w * D)], pe_v)

            def drain(i, carry):
                pltpu.make_async_copy(
                    tab_t_hbm.at[:, pl.ds(0, _BLK)], blocks_v.at[i], sem
                ).wait()
                return carry

            lax.fori_loop(0, b_per_w, drain, 0)
            lane = lax.iota(jnp.int32, _LANES)

            def select(i, carry):
                col_b = jnp.full((_LANES,), _row_at(i) & (_BLK - 1), jnp.int32)
                sel_i = jnp.full((_LANES,), i, jnp.int32)

                def chunk(j, c2):
                    s = pl.ds(i * D + j * _LANES, _LANES)
                    val = plsc.load_gather(
                        blocks_v, [sel_i, j * _LANES + lane, col_b])
                    rows_v[s] = val + pe_v[s]
                    return c2

                lax.fori_loop(0, D // _LANES, chunk, 0)
                return carry

            lax.fori_loop(0, b_per_w, select, 0)
            pltpu.sync_copy(rows_v, out_hbm.at[pl.ds(base * D, b_per_w * D)])

    return sc_embed


def kernel(x, table):
    pe = _pe_np(_CONTEXT_WINDOW, _EMBEDDING_DIM).reshape(-1)
    out = _build_sc_call(_CONTEXT_WINDOW, _EMBEDDING_DIM, 8)(
        x, table.T, jnp.asarray(pe))
    return out.reshape(_CONTEXT_WINDOW, _EMBEDDING_DIM)


# speedup vs baseline: 25.0602x; 1.0249x over previous
import functools

import numpy as np
import jax
import jax.numpy as jnp
from jax import lax
from jax.experimental import pallas as pl
from jax.experimental.pallas import tpu as pltpu
from jax.experimental.pallas import tpu_sc as plsc

_CONTEXT_WINDOW = 200
_EMBEDDING_DIM = 64
_LANES = 16
_BLK = 128


def _pe_np(context_window, embedding_dim):
    pos = np.arange(context_window, dtype=np.float32)[:, None]
    i = np.arange(embedding_dim, dtype=np.float32)[None, :]
    angle = pos / np.power(10000.0, i / embedding_dim)
    pe = np.where((np.arange(embedding_dim)[None, :] % 2) == 0,
                  np.sin(angle), np.cos(angle))
    return pe.astype(np.float32)


@functools.lru_cache(maxsize=None)
def _build_sc_call(B, D, b_per_w):
    mesh = plsc.VectorSubcoreMesh(core_axis_name="c", subcore_axis_name="s")
    info = plsc.get_sparse_core_info()
    nc = info.num_cores
    n_active = B // b_per_w

    @functools.partial(
        pl.kernel,
        mesh=mesh,
        out_type=jax.ShapeDtypeStruct((B * D,), jnp.float32),
        scratch_types=[
            pltpu.VMEM((2 * _LANES,), jnp.int32),
            pltpu.VMEM((b_per_w, D, _BLK), jnp.float32),
            pltpu.VMEM((b_per_w * D,), jnp.float32),
            pltpu.VMEM((b_per_w * D,), jnp.float32),
            pltpu.SemaphoreType.DMA,
            pltpu.SemaphoreType.DMA,
        ],
        compiler_params=pltpu.CompilerParams(needs_layout_passes=False),
    )
    def sc_embed(x_hbm, tab_t_hbm, pe_hbm, out_hbm, idx_v, blocks_v, rows_v,
                 pe_v, sem, sem_pe):
        wid = lax.axis_index("s") * nc + lax.axis_index("c")

        @pl.when(wid < n_active)
        def _():
            base = wid * b_per_w
            pe_cp = pltpu.async_copy(
                pe_hbm.at[pl.ds(base * D, b_per_w * D)], pe_v, sem_pe)
            pltpu.sync_copy(x_hbm.at[pl.ds(base, b_per_w)],
                            idx_v.at[pl.ds(0, b_per_w)])

            def _row_at(i):
                return idx_v[pl.ds(i, _LANES)][0]

            def issue(i, carry):
                row = _row_at(i)
                col = row & (_BLK - 1)
                blk = pl.multiple_of(row - col, _BLK)
                pltpu.async_copy(
                    tab_t_hbm.at[:, pl.ds(blk, _BLK)], blocks_v.at[i], sem)
                return carry

            lax.fori_loop(0, b_per_w, issue, 0)
            pe_cp.wait()
            lane = lax.iota(jnp.int32, _LANES)

            def select(i, carry):
                pltpu.make_async_copy(
                    tab_t_hbm.at[:, pl.ds(0, _BLK)], blocks_v.at[i], sem
                ).wait()
                col_b = jnp.full((_LANES,), _row_at(i) & (_BLK - 1), jnp.int32)
                sel_i = jnp.full((_LANES,), i, jnp.int32)

                def chunk(j, c2):
                    s = pl.ds(i * D + j * _LANES, _LANES)
                    val = plsc.load_gather(
                        blocks_v, [sel_i, j * _LANES + lane, col_b])
                    rows_v[s] = val + pe_v[s]
                    return c2

                lax.fori_loop(0, D // _LANES, chunk, 0)
                return carry

            lax.fori_loop(0, b_per_w, select, 0)
            pltpu.sync_copy(rows_v, out_hbm.at[pl.ds(base * D, b_per_w * D)])

    return sc_embed


def kernel(x, table):
    pe = _pe_np(_CONTEXT_WINDOW, _EMBEDDING_DIM).reshape(-1)
    out = _build_sc_call(_CONTEXT_WINDOW, _EMBEDDING_DIM, 8)(
        x, table.T, jnp.asarray(pe))
    return out.reshape(_CONTEXT_WINDOW, _EMBEDDING_DIM)
